# baseline (speedup 1.0000x reference)
import functools

import jax
import jax.numpy as jnp
from jax import lax
from jax.experimental import pallas as pl
from jax.experimental.pallas import tpu as pltpu
from jax.experimental.pallas import tpu_sc as plsc

_info = plsc.get_sparse_core_info()
_NC, _NS = _info.num_cores, _info.num_subcores
_NW = _NC * _NS

_CHUNK = 128
_NBUF = 4
_NROW = 2


@functools.lru_cache(maxsize=None)
def _build_gather(n_tokens: int, embed: int, seq_l: int):
    b_per_w = n_tokens // _NW
    n_chunks = b_per_w // _CHUNK
    rounds = n_chunks // _NBUF
    n_b = n_tokens // seq_l
    tb_n = n_b // _CHUNK
    eh = embed // 8
    assert n_tokens % _NW == 0 and b_per_w % _CHUNK == 0
    assert n_chunks % _NBUF == 0 and rounds >= 2 and embed == 64
    mesh = plsc.VectorSubcoreMesh(core_axis_name="c", subcore_axis_name="s")

    @functools.partial(
        pl.kernel,
        mesh=mesh,
        out_type=jax.ShapeDtypeStruct((seq_l * eh, tb_n, 8, _CHUNK),
                                      jnp.float32),
        scratch_types=[
            pltpu.VMEM((b_per_w,), jnp.int32),
            pltpu.VMEM((_NROW, _CHUNK), jnp.int32),
            pltpu.VMEM((_NROW, _CHUNK, embed + 16), jnp.float32),
            pltpu.VMEM((_NBUF, eh, 1, 8, _CHUNK + 1), jnp.float32),
        ] + [pltpu.SemaphoreType.DMA] * (_NROW + _NBUF),
        compiler_params=pltpu.CompilerParams(
            use_tc_tiling_on_sc=False, needs_layout_passes=False),
    )
    def gather_kernel(idx_hbm, table_hbm, out_hbm, idx_all, gidx_v, rows_v,
                      buf_v, *sems):
        gsem, osem = sems[:_NROW], sems[_NROW:]
        wid = lax.axis_index("s") * _NC + lax.axis_index("c")
        base = wid * b_per_w
        base_blk = wid * n_chunks
        pltpu.sync_copy(idx_hbm.at[pl.ds(base, b_per_w)], idx_all)
        iota = lax.iota(jnp.int32, 16)
        zero = iota * 0

        def prep_gidx(i, rb):
            ioff = i * _CHUNK

            @plsc.parallel_loop(0, _CHUNK // 16, unroll=2)
            def _(cg):
                gidx_v[rb, pl.ds(cg * 16, 16)] = (
                    idx_all[pl.ds(ioff + cg * 16, 16)])

        def start_gather(rb):
            pltpu.async_copy(table_hbm.at[gidx_v.at[rb]], rows_v.at[rb],
                             gsem[rb])

        def wait_gather(rb):
            pltpu.make_async_copy(
                table_hbm.at[gidx_v.at[rb]], rows_v.at[rb], gsem[rb]).wait()

        ihi = iota >> 3
        ilo = iota & 7

        def permute(i, rb, b, unroll=2):
            @plsc.parallel_loop(0, _CHUNK, unroll=unroll)
            def _(c):
                for k in range(embed // 16):
                    vals = rows_v[rb, c, pl.ds(k * 16, 16)]
                    plsc.store_scatter(
                        buf_v.at[b], [ihi + 2 * k, zero, ilo, zero + c],
                        vals)

        def start_store(i, b):
            blk = base_blk + i
            l8 = (blk // tb_n) * eh
            tb = blk % tb_n
            pltpu.async_copy(
                buf_v.at[b, :, :, :, pl.ds(0, _CHUNK)],
                out_hbm.at[pl.ds(l8, eh), pl.ds(tb, 1)], osem[b])

        def wait_store(b):
            pltpu.make_async_copy(
                buf_v.at[b, :, :, :, pl.ds(0, _CHUNK)],
                out_hbm.at[pl.ds(0, eh), pl.ds(0, 1)],
                osem[b]).wait()

        for b in range(_NBUF):
            prep_gidx(b, b % _NROW)
            start_gather(b % _NROW)
            if b >= 1:
                wait_gather((b - 1) % _NROW)
                permute(b - 1, (b - 1) % _NROW, b - 1, unroll=1)
                start_store(b - 1, b - 1)

        def round_body(r, carry):
            i0 = r * _NBUF
            for b in range(_NBUF):
                rb = b % _NROW
                prev_rb = (b - 1) % _NROW
                prev_b = (b - 1) % _NBUF
                wait_store(b)
                prep_gidx(i0 + b, rb)
                start_gather(rb)
                wait_gather(prev_rb)
                permute(i0 + b - 1, prev_rb, prev_b)
                start_store(i0 + b - 1, prev_b)
            return carry

        lax.fori_loop(1, rounds, round_body, 0)

        wait_gather((n_chunks - 1) % _NROW)
        permute(n_chunks - 1, (n_chunks - 1) % _NROW, _NBUF - 1, unroll=1)
        start_store(n_chunks - 1, _NBUF - 1)
        for b in range(_NBUF):
            wait_store(b)

    return gather_kernel


def kernel(inputtokens, table):
    b, l = inputtokens.shape
    v, e = table.shape
    flat = inputtokens.T.reshape(-1).astype(jnp.int32)
    table_p = jnp.pad(table, ((0, 0), (0, 16)))
    out = _build_gather(b * l, e, l)(flat, table_p)
    return (out.reshape(l, 8, b // 128, 8, 128)
               .transpose(2, 4, 0, 1, 3).reshape(b, l, e))


# speedup vs baseline: 1.7849x; 1.7849x over previous
import functools

import jax
import jax.numpy as jnp
from jax import lax
from jax.experimental import pallas as pl
from jax.experimental.pallas import tpu as pltpu
from jax.experimental.pallas import tpu_sc as plsc

_info = plsc.get_sparse_core_info()
_NC, _NS = _info.num_cores, _info.num_subcores
_NW = _NC * _NS

_CHUNK = 128
_NBUF = 4
_NROW = 2


@functools.lru_cache(maxsize=None)
def _build_gather(n_tokens: int, embed: int, seq_l: int):
    b_per_w = n_tokens // _NW
    n_chunks = b_per_w // _CHUNK
    rounds = n_chunks // _NBUF
    n_b = n_tokens // seq_l
    tb_n = n_b // _CHUNK
    eh = embed // 8
    assert n_tokens % _NW == 0 and b_per_w % _CHUNK == 0
    assert n_chunks % _NBUF == 0 and rounds >= 2 and embed == 64
    mesh = plsc.VectorSubcoreMesh(core_axis_name="c", subcore_axis_name="s")

    @functools.partial(
        pl.kernel,
        mesh=mesh,
        out_type=jax.ShapeDtypeStruct((seq_l * eh, tb_n, 8, _CHUNK),
                                      jnp.float32),
        scratch_types=[
            pltpu.VMEM((b_per_w,), jnp.int32),
            pltpu.VMEM((_NROW, _CHUNK), jnp.int32),
            pltpu.VMEM((_NROW, _CHUNK, embed), jnp.float32),
            pltpu.VMEM((_NBUF, eh, 1, 8, _CHUNK + 1), jnp.float32),
        ] + [pltpu.SemaphoreType.DMA] * (_NROW + _NBUF),
        compiler_params=pltpu.CompilerParams(
            use_tc_tiling_on_sc=False, needs_layout_passes=False),
    )
    def gather_kernel(idx_hbm, table_hbm, out_hbm, idx_all, gidx_v, rows_v,
                      buf_v, *sems):
        gsem, osem = sems[:_NROW], sems[_NROW:]
        wid = lax.axis_index("s") * _NC + lax.axis_index("c")
        base = wid * b_per_w
        base_blk = wid * n_chunks
        pltpu.sync_copy(idx_hbm.at[pl.ds(base, b_per_w)], idx_all)
        iota = lax.iota(jnp.int32, 16)
        zero = iota * 0

        def prep_gidx(i, rb):
            ioff = i * _CHUNK

            @plsc.parallel_loop(0, _CHUNK // 16, unroll=2)
            def _(cg):
                gidx_v[rb, pl.ds(cg * 16, 16)] = (
                    idx_all[pl.ds(ioff + cg * 16, 16)])

        def start_gather(rb):
            pltpu.async_copy(table_hbm.at[gidx_v.at[rb]], rows_v.at[rb],
                             gsem[rb])

        def wait_gather(rb):
            pltpu.make_async_copy(
                table_hbm.at[gidx_v.at[rb]], rows_v.at[rb], gsem[rb]).wait()

        ihi = iota >> 3
        ilo = iota & 7

        def permute(i, rb, b, unroll=2):
            @plsc.parallel_loop(0, _CHUNK, unroll=unroll)
            def _(c):
                for k in range(embed // 16):
                    vals = rows_v[rb, c, pl.ds(k * 16, 16)]
                    plsc.store_scatter(
                        buf_v.at[b], [ihi + 2 * k, zero, ilo, zero + c],
                        vals)

        def start_store(i, b):
            blk = base_blk + i
            l8 = (blk // tb_n) * eh
            tb = blk % tb_n
            pltpu.async_copy(
                buf_v.at[b, :, :, :, pl.ds(0, _CHUNK)],
                out_hbm.at[pl.ds(l8, eh), pl.ds(tb, 1)], osem[b])

        def wait_store(b):
            pltpu.make_async_copy(
                buf_v.at[b, :, :, :, pl.ds(0, _CHUNK)],
                out_hbm.at[pl.ds(0, eh), pl.ds(0, 1)],
                osem[b]).wait()

        for b in range(_NBUF):
            prep_gidx(b, b % _NROW)
            start_gather(b % _NROW)
            if b >= 1:
                wait_gather((b - 1) % _NROW)
                permute(b - 1, (b - 1) % _NROW, b - 1, unroll=1)
                start_store(b - 1, b - 1)

        def round_body(r, carry):
            i0 = r * _NBUF
            for b in range(_NBUF):
                rb = b % _NROW
                prev_rb = (b - 1) % _NROW
                prev_b = (b - 1) % _NBUF
                wait_store(b)
                prep_gidx(i0 + b, rb)
                start_gather(rb)
                wait_gather(prev_rb)
                permute(i0 + b - 1, prev_rb, prev_b)
                start_store(i0 + b - 1, prev_b)
            return carry

        lax.fori_loop(1, rounds, round_body, 0)

        wait_gather((n_chunks - 1) % _NROW)
        permute(n_chunks - 1, (n_chunks - 1) % _NROW, _NBUF - 1, unroll=1)
        start_store(n_chunks - 1, _NBUF - 1)
        for b in range(_NBUF):
            wait_store(b)

    return gather_kernel


def kernel(inputtokens, table):
    b, l = inputtokens.shape
    v, e = table.shape
    flat = inputtokens.T.reshape(-1).astype(jnp.int32) * 2
    table_p = jnp.pad(table, ((0, 0), (0, e))).reshape(2 * v, e)
    out = _build_gather(b * l, e, l)(flat, table_p)
    return (out.reshape(l, 8, b // 128, 8, 128)
               .transpose(2, 4, 0, 1, 3).reshape(b, l, e))
